# baseline (speedup 1.0000x reference)
import functools

import jax
import jax.numpy as jnp
from jax.experimental import pallas as pl
from jax.experimental.pallas import tpu as pltpu

B, S, D, E, C = 4, 2048, 4096, 64, 32

_S_TILE = 512
_ROWS = 8


def _gate_body(x_ref, r_ref, out_ref):
    xt = x_ref[0]
    logits = jax.lax.dot_general(
        r_ref[...], xt, (((1,), (1,)), ((), ())),
        preferred_element_type=jnp.float32)
    m = jnp.max(logits, axis=0, keepdims=True)
    u = jnp.exp(logits - m)
    z = jnp.sum(u, axis=0, keepdims=True)
    out_ref[0] = u / z


def _topk_body(a_ref, g_ref, i_ref, d_ref):
    vals = a_ref[...]
    iota_s = jax.lax.broadcasted_iota(jnp.int32, (_ROWS, S), 1)
    iota_c = jax.lax.broadcasted_iota(jnp.int32, (_ROWS, C), 1)
    g = jnp.zeros((_ROWS, C), jnp.float32)
    idx = jnp.zeros((_ROWS, C), jnp.int32)
    for c in range(C):
        m = jnp.max(vals, axis=1, keepdims=True)
        am = jnp.min(jnp.where(vals == m, iota_s, S), axis=1, keepdims=True)
        g = jnp.where(iota_c == c, m, g)
        idx = jnp.where(iota_c == c, am, idx)
        vals = jnp.where(iota_s == am, -jnp.inf, vals)
    g_ref[...] = g
    i_ref[...] = idx
    d_ref[...] = (
        idx[:, :, None]
        == jax.lax.broadcasted_iota(jnp.int32, (_ROWS, C, S), 2)
    ).astype(jnp.float32)


@jax.jit
def kernel(x, router):
    aff_t = pl.pallas_call(
        _gate_body,
        grid=(B, S // _S_TILE),
        in_specs=[
            pl.BlockSpec((1, _S_TILE, D), lambda b, s: (b, s, 0)),
            pl.BlockSpec((E, D), lambda b, s: (0, 0)),
        ],
        out_specs=pl.BlockSpec((1, E, _S_TILE), lambda b, s: (b, 0, s)),
        out_shape=jax.ShapeDtypeStruct((B, E, S), jnp.float32),
    )(x, router)

    rows = B * E
    aff2 = aff_t.reshape(rows, S)
    gating, index, dispatch = pl.pallas_call(
        _topk_body,
        grid=(rows // _ROWS,),
        in_specs=[pl.BlockSpec((_ROWS, S), lambda i: (i, 0))],
        out_specs=[
            pl.BlockSpec((_ROWS, C), lambda i: (i, 0)),
            pl.BlockSpec((_ROWS, C), lambda i: (i, 0)),
            pl.BlockSpec((_ROWS, C, S), lambda i: (i, 0, 0)),
        ],
        out_shape=[
            jax.ShapeDtypeStruct((rows, C), jnp.float32),
            jax.ShapeDtypeStruct((rows, C), jnp.int32),
            jax.ShapeDtypeStruct((rows, C, S), jnp.float32),
        ],
    )(aff2)

    return (gating.reshape(B, E, C),
            dispatch.reshape(B, E, C, S),
            index.reshape(B, E, C))


# speedup vs baseline: 2.7589x; 2.7589x over previous
import jax
import jax.numpy as jnp
from jax.experimental import pallas as pl

B, S, D, E, C = 4, 2048, 4096, 64, 32

_S_TILE = 512
_TK_ROWS = 64
_DP_ROWS = 8


def _gate_body(x_ref, r_ref, out_ref):
    xt = x_ref[0]
    logits = jax.lax.dot_general(
        r_ref[...], xt, (((1,), (1,)), ((), ())),
        preferred_element_type=jnp.float32)
    m = jnp.max(logits, axis=0, keepdims=True)
    u = jnp.exp(logits - m)
    z = jnp.sum(u, axis=0, keepdims=True)
    out_ref[0] = u / z


def _topk_body(a_ref, g_ref, i_ref):
    vals = a_ref[...]
    iota_s = jax.lax.broadcasted_iota(jnp.int32, (_TK_ROWS, S), 1)
    iota_c = jax.lax.broadcasted_iota(jnp.int32, (_TK_ROWS, C), 1)
    g = jnp.zeros((_TK_ROWS, C), jnp.float32)
    idx = jnp.zeros((_TK_ROWS, C), jnp.int32)
    for c in range(C):
        m = jnp.max(vals, axis=1, keepdims=True)
        am = jnp.min(jnp.where(vals == m, iota_s, S), axis=1, keepdims=True)
        g = jnp.where(iota_c == c, m, g)
        idx = jnp.where(iota_c == c, am, idx)
        vals = jnp.where(iota_s == am, -jnp.inf, vals)
    g_ref[...] = g
    i_ref[...] = idx


def _dispatch_body(i_ref, d_ref):
    idx = i_ref[...]
    d_ref[...] = (
        idx[:, :, None]
        == jax.lax.broadcasted_iota(jnp.int32, (_DP_ROWS, C, S), 2)
    ).astype(jnp.float32)


@jax.jit
def kernel(x, router):
    aff_t = pl.pallas_call(
        _gate_body,
        grid=(B, S // _S_TILE),
        in_specs=[
            pl.BlockSpec((1, _S_TILE, D), lambda b, s: (b, s, 0)),
            pl.BlockSpec((E, D), lambda b, s: (0, 0)),
        ],
        out_specs=pl.BlockSpec((1, E, _S_TILE), lambda b, s: (b, 0, s)),
        out_shape=jax.ShapeDtypeStruct((B, E, S), jnp.float32),
    )(x, router)

    rows = B * E
    aff2 = aff_t.reshape(rows, S)
    gating, index = pl.pallas_call(
        _topk_body,
        grid=(rows // _TK_ROWS,),
        in_specs=[pl.BlockSpec((_TK_ROWS, S), lambda i: (i, 0))],
        out_specs=[
            pl.BlockSpec((_TK_ROWS, C), lambda i: (i, 0)),
            pl.BlockSpec((_TK_ROWS, C), lambda i: (i, 0)),
        ],
        out_shape=[
            jax.ShapeDtypeStruct((rows, C), jnp.float32),
            jax.ShapeDtypeStruct((rows, C), jnp.int32),
        ],
    )(aff2)

    dispatch = pl.pallas_call(
        _dispatch_body,
        grid=(rows // _DP_ROWS,),
        in_specs=[pl.BlockSpec((_DP_ROWS, C), lambda i: (i, 0))],
        out_specs=pl.BlockSpec((_DP_ROWS, C, S), lambda i: (i, 0, 0)),
        out_shape=jax.ShapeDtypeStruct((rows, C, S), jnp.float32),
    )(index)

    return (gating.reshape(B, E, C),
            dispatch.reshape(B, E, C, S),
            index.reshape(B, E, C))
